# baseline (speedup 1.0000x reference)
import functools

import jax
import jax.numpy as jnp
from jax import lax
from jax.experimental import pallas as pl
from jax.experimental.pallas import tpu as pltpu
from jax.experimental.pallas import tpu_sc as plsc

_NC = 2
_NS = 16
_NW = _NC * _NS
_LANES = 16


def _prep_body(c_ref, w1_ref, b1_ref, w2_ref, b2_ref, t1_ref, t2_ref):
    c = c_ref[...]
    d = c.shape[1]
    t1a = jnp.dot(c, w1_ref[:d, :], preferred_element_type=jnp.float32) + b1_ref[...]
    t1b = jnp.dot(c, w1_ref[d:, :], preferred_element_type=jnp.float32)
    t2 = jnp.dot(c, w2_ref[...], preferred_element_type=jnp.float32) + b2_ref[...]
    t1_ref[...] = jnp.concatenate([t1a, t1b], axis=1)
    t2_ref[...] = jnp.concatenate([t2, t2], axis=1)


def _make_tables(constants, w1, b1, w2, b2):
    v, d = constants.shape
    a = w1.shape[1]
    bm = 2000
    assert v % bm == 0
    grid = (v // bm,)
    ts = jax.ShapeDtypeStruct((v, 2 * a), jnp.float32)
    return pl.pallas_call(
        _prep_body,
        grid=grid,
        in_specs=[
            pl.BlockSpec((bm, d), lambda i: (i, 0)),
            pl.BlockSpec((2 * d, a), lambda i: (0, 0)),
            pl.BlockSpec((1, a), lambda i: (0, 0)),
            pl.BlockSpec((d, a), lambda i: (0, 0)),
            pl.BlockSpec((1, a), lambda i: (0, 0)),
        ],
        out_specs=[
            pl.BlockSpec((bm, 2 * a), lambda i: (i, 0)),
            pl.BlockSpec((bm, 2 * a), lambda i: (i, 0)),
        ],
        out_shape=[ts, ts],
    )(constants, w1, b1.reshape(1, a), w2, b2.reshape(1, a))


def _pick_chunk(n):
    for c in (200, 100, 40, 8):
        if n % c == 0:
            return c
    raise ValueError(f"no chunk size for {n}")


def _gather_embed(t1, t2d, i1, i2, a):
    n1 = i1.shape[0] // 2
    n2 = i2.shape[0]
    c1 = _pick_chunk(n1)
    c2 = _pick_chunk(n2)
    assert c1 == c2
    g1 = n1 // c1
    g2 = n2 // c2
    k1 = -(-g1 // _NW)
    k2 = -(-g2 // _NW)
    nv = a // _LANES

    mesh = plsc.VectorSubcoreMesh(
        core_axis_name="c", subcore_axis_name="s", num_cores=_NC, num_subcores=_NS
    )

    @functools.partial(
        pl.kernel,
        out_type=jax.ShapeDtypeStruct((n1 + n2, a), jnp.float32),
        mesh=mesh,
        compiler_params=pltpu.CompilerParams(use_tc_tiling_on_sc=True),
        scratch_types=[
            pltpu.VMEM((2 * c1,), jnp.int32),
            pltpu.VMEM((2 * c1, 2 * a), jnp.float32),
            pltpu.VMEM((c1, a), jnp.float32),
            pltpu.SemaphoreType.DMA,
        ],
    )
    def sc_kernel(t1_h, t2d_h, i1_h, i2_h, out_h, iv, rows, obuf, sem):
        wid = lax.axis_index("s") * _NC + lax.axis_index("c")

        def p1_chunk(k, carry):
            g = wid + k * _NW

            @pl.when(g < g1)
            def _():
                base = g * c1
                pltpu.sync_copy(i1_h.at[pl.ds(2 * base, 2 * c1)], iv)
                pltpu.async_copy(t1_h.at[iv], rows, sem).wait()

                def pack_row(n, carry2):
                    for j in range(nv):
                        x = rows[2 * n, pl.ds(j * _LANES, _LANES)]
                        y = rows[2 * n + 1, pl.ds(a + j * _LANES, _LANES)]
                        obuf[n, pl.ds(j * _LANES, _LANES)] = x + y
                    return carry2

                lax.fori_loop(0, c1, pack_row, 0)
                pltpu.sync_copy(obuf, out_h.at[pl.ds(base, c1)])

            return carry

        lax.fori_loop(0, k1, p1_chunk, 0)

        def p2_chunk(k, carry):
            g = wid + k * _NW

            @pl.when(g < g2)
            def _():
                base = g * c2
                pltpu.sync_copy(i2_h.at[pl.ds(base, c2)], iv.at[pl.ds(0, c2)])
                pltpu.async_copy(t2d_h.at[iv.at[pl.ds(0, c2)]], rows.at[pl.ds(0, c2)], sem).wait()

                def pack_row(n, carry2):
                    for j in range(nv):
                        obuf[n, pl.ds(j * _LANES, _LANES)] = rows[
                            n, pl.ds(j * _LANES, _LANES)
                        ]
                    return carry2

                lax.fori_loop(0, c2, pack_row, 0)
                pltpu.sync_copy(obuf, out_h.at[pl.ds(n1 + base, c2)])

            return carry

        lax.fori_loop(0, k2, p2_chunk, 0)

    return sc_kernel(t1, t2d, i1, i2)


def kernel(constants_entity, tuples_p1, tuples_p2, W_p1, b_p1, W_p2, b_p2):
    t1, t2d = _make_tables(constants_entity, W_p1, b_p1, W_p2, b_p2)
    i1 = tuples_p1.astype(jnp.int32).reshape(-1)
    i2 = tuples_p2.astype(jnp.int32).reshape(-1)
    return _gather_embed(t1, t2d, i1, i2, W_p1.shape[1])


# speedup vs baseline: 1.4826x; 1.4826x over previous
import functools

import jax
import jax.numpy as jnp
from jax import lax
from jax.experimental import pallas as pl
from jax.experimental.pallas import tpu as pltpu
from jax.experimental.pallas import tpu_sc as plsc

_NC = 2
_NS = 16
_NW = _NC * _NS
_LANES = 16
_C = 128


def _prep_body(c_ref, w1_ref, b1_ref, w2_ref, b2_ref, u_ref, *, nb):
    i = pl.program_id(0)
    c = c_ref[...]
    d = c.shape[1]

    @pl.when(i < nb)
    def _():
        t1a = jnp.dot(c, w1_ref[:d, :], preferred_element_type=jnp.float32)
        t1b = jnp.dot(c, w1_ref[d:, :], preferred_element_type=jnp.float32)
        u_ref[...] = jnp.concatenate([t1a + b1_ref[...], t1b], axis=1)

    @pl.when((i >= nb) & (i < 2 * nb))
    def _():
        t2 = jnp.dot(c, w2_ref[...], preferred_element_type=jnp.float32)
        u_ref[...] = jnp.concatenate([t2 + b2_ref[...], t2], axis=1)

    @pl.when(i == 2 * nb)
    def _():
        u_ref[...] = jnp.zeros_like(u_ref)


def _make_table(constants, w1, b1, w2, b2):
    v, d = constants.shape
    a = w1.shape[1]
    bm = 2000
    assert v % bm == 0
    nb = v // bm
    grid = (2 * nb + 1,)
    return pl.pallas_call(
        functools.partial(_prep_body, nb=nb),
        grid=grid,
        in_specs=[
            pl.BlockSpec((bm, d), lambda i: (lax.rem(i, nb), 0)),
            pl.BlockSpec((2 * d, a), lambda i: (0, 0)),
            pl.BlockSpec((1, a), lambda i: (0, 0)),
            pl.BlockSpec((d, a), lambda i: (0, 0)),
            pl.BlockSpec((1, a), lambda i: (0, 0)),
        ],
        out_specs=pl.BlockSpec((bm, 2 * a), lambda i: (i, 0)),
        out_shape=jax.ShapeDtypeStruct((2 * v + bm, 2 * a), jnp.float32),
    )(constants, w1, b1.reshape(1, a), w2, b2.reshape(1, a))


def _gather_embed(u, idxa, idxb, a, n1):
    ntot = idxa.shape[0]
    assert ntot % 2 == 0
    hc = _C // 2
    nch = ntot // _C
    tail = ntot - nch * _C
    g1b = -(-n1 // _C)
    kf = nch // _NW
    if kf % 2 == 1:
        kf -= 1
    nv = a // _LANES

    mesh = plsc.VectorSubcoreMesh(
        core_axis_name="c", subcore_axis_name="s", num_cores=_NC, num_subcores=_NS
    )

    @functools.partial(
        pl.kernel,
        out_type=jax.ShapeDtypeStruct((ntot // 2, 2 * a), jnp.float32),
        mesh=mesh,
        compiler_params=pltpu.CompilerParams(
            use_tc_tiling_on_sc=True, needs_layout_passes=False
        ),
        scratch_types=[
            pltpu.VMEM((_C,), jnp.int32),
            pltpu.VMEM((_C,), jnp.int32),
            pltpu.VMEM((_C,), jnp.int32),
            pltpu.VMEM((_C,), jnp.int32),
            pltpu.VMEM((_C, 2 * a), jnp.float32),
            pltpu.VMEM((_C, 2 * a), jnp.float32),
            pltpu.VMEM((_C, 2 * a), jnp.float32),
            pltpu.VMEM((_C, 2 * a), jnp.float32),
            pltpu.VMEM((hc, 2 * a), jnp.float32),
            pltpu.VMEM((hc, 2 * a), jnp.float32),
            pltpu.SemaphoreType.DMA,
            pltpu.SemaphoreType.DMA,
            pltpu.SemaphoreType.DMA,
            pltpu.SemaphoreType.DMA,
            pltpu.SemaphoreType.DMA,
            pltpu.SemaphoreType.DMA,
        ],
    )
    def sc_kernel(u_h, ia_h, ib_h, out_h, iva0, iva1, ivb0, ivb1, ba0, ba1,
                  bb0, bb1, ob0, ob1, si0, si1, sg0, sg1, so0, so1):
        wid = lax.axis_index("s") * _NC + lax.axis_index("c")
        iva, ivb = (iva0, iva1), (ivb0, ivb1)
        ba, bb = (ba0, ba1), (bb0, bb1)
        ob = (ob0, ob1)
        si, sg, so = (si0, si1), (sg0, sg1), (so0, so1)

        def gofk(k):
            return wid + k * _NW

        def start_idx(k, s):
            base = gofk(k) * _C
            pltpu.async_copy(ia_h.at[pl.ds(base, _C)], iva[s], si[s])
            pltpu.async_copy(ib_h.at[pl.ds(base, _C)], ivb[s], si[s])

        def wait_idx(s):
            pltpu.make_async_copy(ia_h.at[pl.ds(0, _C)], iva[s], si[s]).wait()
            pltpu.make_async_copy(ib_h.at[pl.ds(0, _C)], ivb[s], si[s]).wait()

        def start_gather(k, s):
            g = gofk(k)

            @pl.when(g < g1b)
            def _():
                pltpu.async_copy(u_h.at[iva[s]], ba[s], sg[s])
                pltpu.async_copy(u_h.at[ivb[s]], bb[s], sg[s])

            @pl.when(g >= g1b)
            def _():
                pltpu.async_copy(u_h.at[iva[s]], ba[s], sg[s])

        def wait_gather(k, s):
            g = gofk(k)
            pltpu.make_async_copy(u_h.at[iva[s]], ba[s], sg[s]).wait()

            @pl.when(g < g1b)
            def _():
                pltpu.make_async_copy(u_h.at[ivb[s]], bb[s], sg[s]).wait()

        def pack(k, s):
            g = gofk(k)

            @pl.when(g < g1b)
            def _():
                @plsc.parallel_loop(0, hc, unroll=4)
                def _(m):
                    for h in range(2):
                        n = 2 * m + h
                        for j in range(nv):
                            x = ba[s][n, pl.ds(j * _LANES, _LANES)]
                            y = bb[s][n, pl.ds(a + j * _LANES, _LANES)]
                            ob[s][m, pl.ds(h * a + j * _LANES, _LANES)] = x + y

            @pl.when(g >= g1b)
            def _():
                @plsc.parallel_loop(0, hc, unroll=4)
                def _(m):
                    for h in range(2):
                        n = 2 * m + h
                        for j in range(nv):
                            x = ba[s][n, pl.ds(j * _LANES, _LANES)]
                            ob[s][m, pl.ds(h * a + j * _LANES, _LANES)] = x

        def start_out(k, s):
            base2 = gofk(k) * hc
            pltpu.async_copy(ob[s], out_h.at[pl.ds(base2, hc)], so[s])

        def wait_out(s):
            pltpu.make_async_copy(ob[s], out_h.at[pl.ds(0, hc)], so[s]).wait()

        def half(k, s, sother):
            wait_idx(s)
            start_gather(k, s)

            @pl.when(k >= 1)
            def _():
                wait_gather(k - 1, sother)

                @pl.when(k >= 3)
                def _():
                    wait_out(sother)

                pack(k - 1, sother)
                start_out(k - 1, sother)

            @pl.when(gofk(k + 1) < nch)
            def _():
                start_idx(k + 1, sother)

        start_idx(0, 0)

        def body(kk, carry):
            half(2 * kk, 0, 1)
            half(2 * kk + 1, 1, 0)
            return carry

        lax.fori_loop(0, kf // 2, body, 0)

        if kf >= 1:
            wait_gather(kf - 1, 1)
            if kf >= 3:
                wait_out(1)
            pack(kf - 1, 1)
            start_out(kf - 1, 1)

        for extra, s in ((kf, 0), (kf + 1, 1)):
            @pl.when(gofk(extra) < nch)
            def _(extra=extra, s=s):
                wait_idx(s)
                start_gather(extra, s)
                wait_gather(extra, s)
                wait_out(s)
                pack(extra, s)
                start_out(extra, s)

        wait_out(0)
        wait_out(1)

        if tail > 0:
            tbase = nch * _C
            t2 = tail // 2

            @pl.when(wid == 0)
            def _():
                pltpu.sync_copy(ia_h.at[pl.ds(tbase, tail)],
                                iva[0].at[pl.ds(0, tail)])
                pltpu.async_copy(u_h.at[iva[0].at[pl.ds(0, tail)]],
                                 ba[0].at[pl.ds(0, tail)], sg[0]).wait()
                if tbase < n1:
                    pltpu.sync_copy(ib_h.at[pl.ds(tbase, tail)],
                                    ivb[0].at[pl.ds(0, tail)])
                    pltpu.async_copy(u_h.at[ivb[0].at[pl.ds(0, tail)]],
                                     bb[0].at[pl.ds(0, tail)], sg[0]).wait()

                    @plsc.parallel_loop(0, t2, unroll=4)
                    def _(m):
                        for h in range(2):
                            n = 2 * m + h
                            for j in range(nv):
                                x = ba[0][n, pl.ds(j * _LANES, _LANES)]
                                y = bb[0][n, pl.ds(a + j * _LANES, _LANES)]
                                ob[0][m, pl.ds(h * a + j * _LANES, _LANES)] = x + y
                else:
                    @plsc.parallel_loop(0, t2, unroll=4)
                    def _(m):
                        for h in range(2):
                            n = 2 * m + h
                            for j in range(nv):
                                x = ba[0][n, pl.ds(j * _LANES, _LANES)]
                                ob[0][m, pl.ds(h * a + j * _LANES, _LANES)] = x

                pltpu.sync_copy(ob[0].at[pl.ds(0, t2)],
                                out_h.at[pl.ds(nch * hc, t2)])

    return sc_kernel(u, idxa, idxb)


def kernel(constants_entity, tuples_p1, tuples_p2, W_p1, b_p1, W_p2, b_p2):
    v = constants_entity.shape[0]
    a = W_p1.shape[1]
    n1 = tuples_p1.shape[0]
    n2 = tuples_p2.shape[0]
    u = _make_table(constants_entity, W_p1, b_p1, W_p2, b_p2)
    zrow = 2 * v
    idxa = jnp.concatenate([
        tuples_p1[:, 0].astype(jnp.int32),
        tuples_p2[:, 0].astype(jnp.int32) + v,
    ])
    idxb = jnp.concatenate([
        tuples_p1[:, 1].astype(jnp.int32),
        jnp.full((n2,), zrow, jnp.int32),
    ])
    out2 = _gather_embed(u, idxa, idxb, a, n1)
    return out2.reshape(n1 + n2, a)
